# Initial kernel scaffold; baseline (speedup 1.0000x reference)
import jax
import jax.numpy as jnp
from jax.experimental import pallas as pl


def kernel(z):
    raise NotImplementedError("write your pallas kernel here")


# speedup vs baseline: 14.3309x; 14.3309x over previous
import functools

import jax
import jax.numpy as jnp
from jax.experimental import pallas as pl
from jax.experimental.pallas import tpu as pltpu

_TAU = 0.9


def _tc_body(z_ref, o_ref):
    _IMIN = jnp.int32(-2147483648)
    z = z_ref[...]
    m = jnp.max(z, axis=-1, keepdims=True)
    e = jnp.exp(z - m)
    s = jnp.sum(e, axis=-1, keepdims=True)
    t = jnp.float32(_TAU) * s

    b = jax.lax.bitcast_convert_type(z, jnp.int32)
    ukey = jnp.where(b >= 0, b | _IMIN, ~b)
    skey = ukey ^ _IMIN

    k0 = jnp.zeros_like(m, dtype=jnp.int32)

    def step(i, kacc):
        bit = jnp.int32(1) << (31 - i)
        kc = kacc | bit
        ge = skey >= (kc ^ _IMIN)
        h = jnp.sum(jnp.where(ge, e, 0.0), axis=-1, keepdims=True)
        return jnp.where(h >= t, kc, kacc)

    kstar = jax.lax.fori_loop(0, 32, step, k0)

    fbits = jnp.where(kstar < 0, kstar & ~_IMIN, ~kstar)
    vstar = jax.lax.bitcast_convert_type(fbits, jnp.float32)

    g_above = jnp.sum(jnp.where(z > vstar, e, 0.0), axis=-1, keepdims=True)
    estar = jnp.exp(vstar - m)
    j = jnp.ceil((t - g_above) / estar)
    eq = z == vstar
    eqf = eq.astype(jnp.float32)
    m_cnt = jnp.sum(eqf, axis=-1, keepdims=True)
    idx = jax.lax.broadcasted_iota(jnp.int32, z.shape, 1)

    def tie_select(_):
        i0 = jnp.zeros_like(m, dtype=jnp.int32)

        def tstep(i, iacc):
            ic = iacc | (jnp.int32(1) << (12 - i))
            cnt = jnp.sum(jnp.where(eq & (idx <= ic), 1.0, 0.0),
                          axis=-1, keepdims=True)
            return jnp.where(cnt < j, ic, iacc)

        iacc = jax.lax.fori_loop(0, 13, tstep, i0)
        cnt = jnp.sum(jnp.where(eq & (idx <= iacc), 1.0, 0.0),
                      axis=-1, keepdims=True)
        return jnp.where(cnt < j, iacc + 1, iacc)

    bound = jax.lax.cond(jnp.any(m_cnt > j), tie_select,
                         lambda _: jnp.full_like(m, 8192, dtype=jnp.int32),
                         operand=None)
    mask = (z > vstar) | (eq & (idx <= bound))
    o_ref[...] = jnp.where(mask, jnp.maximum(z, 0.0), 0.0)


@jax.jit
def kernel(z):
    rows, n = z.shape
    br = 16
    grid = rows // br
    return pl.pallas_call(
        _tc_body,
        grid=(grid,),
        in_specs=[pl.BlockSpec((br, n), lambda i: (i, 0))],
        out_specs=pl.BlockSpec((br, n), lambda i: (i, 0)),
        out_shape=jax.ShapeDtypeStruct((rows, n), jnp.float32),
    )(z)
